# baseline (speedup 1.0000x reference)
import functools

import jax
import jax.numpy as jnp
from jax import lax
from jax.experimental import pallas as pl
from jax.experimental.pallas import tpu as pltpu
from jax.experimental.pallas import tpu_sc as plsc

N_CLS = 1000
N_CTX = 16
CTX_DIM = 512
SEQ = 77
SUF = SEQ - 1 - N_CTX
B = 1024
TOK_PAD = 96

NW = 32
BPW = B // NW
CTX_CHUNK = 4
SUF_CHUNK = 2


def _sc_body(pre_n, ctx_n, suf_n, pre_p, ctx_p, suf_p, tok_n, tok_p,
             cls1, cls_c, cls_s,
             out_prompts, out_tok,
             idx_v, idx_c, idx_s, pre_buf, ctx_buf, suf_buf, tok_buf, sem):
    wid = lax.axis_index("s") * 2 + lax.axis_index("c")
    base = wid * BPW

    pltpu.sync_copy(cls1.at[pl.ds(base, BPW)], idx_v)
    pltpu.sync_copy(cls_c.at[wid], idx_c)
    pltpu.sync_copy(cls_s.at[wid], idx_s)

    for s, (pre, ctx, suf, tok) in enumerate(
            [(pre_n, ctx_n, suf_n, tok_n), (pre_p, ctx_p, suf_p, tok_p)]):
        pltpu.async_copy(pre.at[idx_v], pre_buf, sem).wait()
        pltpu.sync_copy(pre_buf, out_prompts.at[s, pl.ds(base, BPW), pl.ds(0, 1), :])

        for j in range(BPW // CTX_CHUNK):
            pltpu.async_copy(ctx.at[idx_c.at[j]], ctx_buf, sem).wait()
            pltpu.sync_copy(
                ctx_buf,
                out_prompts.at[s, pl.ds(base + j * CTX_CHUNK, CTX_CHUNK),
                               pl.ds(1, N_CTX), :])

        for j in range(BPW // SUF_CHUNK):
            pltpu.async_copy(suf.at[idx_s.at[j]], suf_buf, sem).wait()
            pltpu.sync_copy(
                suf_buf,
                out_prompts.at[s, pl.ds(base + j * SUF_CHUNK, SUF_CHUNK),
                               pl.ds(1 + N_CTX, SUF), :])

        pltpu.async_copy(tok.at[idx_v], tok_buf, sem).wait()
        pltpu.sync_copy(tok_buf, out_tok.at[s, pl.ds(base, BPW), :])


_sc_call = functools.partial(
    pl.kernel,
    mesh=plsc.VectorSubcoreMesh(core_axis_name="c", subcore_axis_name="s"),
    compiler_params=pltpu.CompilerParams(use_tc_tiling_on_sc=False),
    out_type=(
        jax.ShapeDtypeStruct((2, B, SEQ, CTX_DIM), jnp.float32),
        jax.ShapeDtypeStruct((2, B, TOK_PAD), jnp.int32),
    ),
    scratch_types=[
        pltpu.VMEM((BPW,), jnp.int32),
        pltpu.VMEM((BPW // CTX_CHUNK, CTX_CHUNK), jnp.int32),
        pltpu.VMEM((BPW // SUF_CHUNK, SUF_CHUNK), jnp.int32),
        pltpu.VMEM((BPW, 1, CTX_DIM), jnp.float32),
        pltpu.VMEM((CTX_CHUNK, N_CTX, CTX_DIM), jnp.float32),
        pltpu.VMEM((SUF_CHUNK, SUF, CTX_DIM), jnp.float32),
        pltpu.VMEM((BPW, TOK_PAD), jnp.int32),
        pltpu.SemaphoreType.DMA,
    ],
)(_sc_body)


def kernel(ctx_pos, ctx_neg, token_prefix_pos, token_suffix_pos,
           token_prefix_neg, token_suffix_neg, tokenized_prompts, cls_id):
    tok_n = jnp.pad(tokenized_prompts[:N_CLS], ((0, 0), (0, TOK_PAD - SEQ)))
    tok_p = jnp.pad(tokenized_prompts[N_CLS:], ((0, 0), (0, TOK_PAD - SEQ)))
    cls_c = cls_id.reshape(NW, BPW // CTX_CHUNK, CTX_CHUNK)
    cls_s = cls_id.reshape(NW, BPW // SUF_CHUNK, SUF_CHUNK)
    prompts, tok = _sc_call(
        token_prefix_neg, ctx_neg, token_suffix_neg,
        token_prefix_pos, ctx_pos, token_suffix_pos,
        tok_n, tok_p, cls_id, cls_c, cls_s)
    return (prompts.reshape(2 * B, SEQ, CTX_DIM),
            tok.reshape(2 * B, TOK_PAD)[:, :SEQ])


# speedup vs baseline: 1.0072x; 1.0072x over previous
import functools

import jax
import jax.numpy as jnp
from jax import lax
from jax.experimental import pallas as pl
from jax.experimental.pallas import tpu as pltpu
from jax.experimental.pallas import tpu_sc as plsc

N_CLS = 1000
N_CTX = 16
CTX_DIM = 512
SEQ = 77
SUF = SEQ - 1 - N_CTX
B = 1024
TOK_PAD = 96

NW = 32
BPW = B // NW
CTX_CHUNK = 2
SUF_CHUNK = 1
NBUF = 2


def _sc_body(pre_n, ctx_n, suf_n, pre_p, ctx_p, suf_p, tok_n, tok_p,
             cls1, cls_c, cls_s,
             out_prompts, out_tok,
             idx_v, idx_c, idx_s, pre_buf, ctx_bufs, suf_bufs, tok_buf,
             gsem_c, gsem_s, wsem_c, wsem_s, sem_pre, sem_tok):
    wid = lax.axis_index("s") * 2 + lax.axis_index("c")
    base = wid * BPW

    pltpu.sync_copy(cls1.at[pl.ds(base, BPW)], idx_v)
    pltpu.sync_copy(cls_c.at[wid], idx_c)
    pltpu.sync_copy(cls_s.at[wid], idx_s)

    tabs = [(pre_n, ctx_n, suf_n, tok_n), (pre_p, ctx_p, suf_p, tok_p)]

    tok_g = [pltpu.async_copy(tabs[s][3].at[idx_v], tok_buf.at[s], sem_tok)
             for s in range(2)]

    def run_ring(table_ix, idx_ref, bufs, chunk, band0, bandw, gsems, wsems):
        nchunks = BPW // chunk
        items = [(s, j) for s in range(2) for j in range(nchunks)]
        gh = [None] * len(items)
        wh = [None] * len(items)
        for k, (s, j) in enumerate(items):
            i = k % NBUF
            if k >= NBUF:
                wh[k - NBUF].wait()
            gh[k] = pltpu.async_copy(tabs[s][table_ix].at[idx_ref.at[j]],
                                     bufs.at[i], gsems[i])
            gh[k].wait()
            wh[k] = pltpu.async_copy(
                bufs.at[i],
                out_prompts.at[s, pl.ds(base + j * chunk, chunk),
                               pl.ds(band0, bandw), :],
                wsems[i])
        for k in range(len(items) - NBUF, len(items)):
            wh[k].wait()

    run_ring(2, idx_s, suf_bufs, SUF_CHUNK, 1 + N_CTX, SUF,
             [gsem_s.at[i] for i in range(NBUF)],
             [wsem_s.at[i] for i in range(NBUF)])
    run_ring(1, idx_c, ctx_bufs, CTX_CHUNK, 1, N_CTX,
             [gsem_c.at[i] for i in range(NBUF)],
             [wsem_c.at[i] for i in range(NBUF)])

    for s in range(2):
        pltpu.async_copy(tabs[s][0].at[idx_v], pre_buf, sem_pre).wait()
        pltpu.async_copy(pre_buf,
                         out_prompts.at[s, pl.ds(base, BPW), pl.ds(0, 1), :],
                         sem_pre).wait()
        tok_g[s].wait()
        pltpu.async_copy(tok_buf.at[s], out_tok.at[s, pl.ds(base, BPW), :],
                         sem_tok).wait()


_sc_call = functools.partial(
    pl.kernel,
    mesh=plsc.VectorSubcoreMesh(core_axis_name="c", subcore_axis_name="s"),
    compiler_params=pltpu.CompilerParams(use_tc_tiling_on_sc=False),
    out_type=(
        jax.ShapeDtypeStruct((2, B, SEQ, CTX_DIM), jnp.float32),
        jax.ShapeDtypeStruct((2, B, TOK_PAD), jnp.int32),
    ),
    scratch_types=[
        pltpu.VMEM((BPW,), jnp.int32),
        pltpu.VMEM((BPW // CTX_CHUNK, CTX_CHUNK), jnp.int32),
        pltpu.VMEM((BPW // SUF_CHUNK, SUF_CHUNK), jnp.int32),
        pltpu.VMEM((BPW, 1, CTX_DIM), jnp.float32),
        pltpu.VMEM((NBUF, CTX_CHUNK, N_CTX, CTX_DIM), jnp.float32),
        pltpu.VMEM((NBUF, SUF_CHUNK, SUF, CTX_DIM), jnp.float32),
        pltpu.VMEM((2, BPW, TOK_PAD), jnp.int32),
        pltpu.SemaphoreType.DMA((NBUF,)),
        pltpu.SemaphoreType.DMA((NBUF,)),
        pltpu.SemaphoreType.DMA((NBUF,)),
        pltpu.SemaphoreType.DMA((NBUF,)),
        pltpu.SemaphoreType.DMA,
        pltpu.SemaphoreType.DMA,
    ],
)(_sc_body)


def kernel(ctx_pos, ctx_neg, token_prefix_pos, token_suffix_pos,
           token_prefix_neg, token_suffix_neg, tokenized_prompts, cls_id):
    tok_n = jnp.pad(tokenized_prompts[:N_CLS], ((0, 0), (0, TOK_PAD - SEQ)))
    tok_p = jnp.pad(tokenized_prompts[N_CLS:], ((0, 0), (0, TOK_PAD - SEQ)))
    cls_c = cls_id.reshape(NW, BPW // CTX_CHUNK, CTX_CHUNK)
    cls_s = cls_id.reshape(NW, BPW // SUF_CHUNK, SUF_CHUNK)
    prompts, tok = _sc_call(
        token_prefix_neg, ctx_neg, token_suffix_neg,
        token_prefix_pos, ctx_pos, token_suffix_pos,
        tok_n, tok_p, cls_id, cls_c, cls_s)
    return (prompts.reshape(2 * B, SEQ, CTX_DIM),
            tok.reshape(2 * B, TOK_PAD)[:, :SEQ])


# speedup vs baseline: 6.3738x; 6.3283x over previous
import functools

import jax
import jax.numpy as jnp
from jax import lax
from jax.experimental import pallas as pl
from jax.experimental.pallas import tpu as pltpu
from jax.experimental.pallas import tpu_sc as plsc

N_CLS = 1000
N_CTX = 16
CTX_DIM = 512
SEQ = 77
SUF = SEQ - 1 - N_CTX
B = 1024
TOK_PAD = 128

NW = 32
BPW = B // NW
NBUF = 4
LOOK = 2


def _sc_body(pre_n, ctx_n, suf_n, pre_p, ctx_p, suf_p, tok_n, tok_p, cls1,
             out_t, out_tok,
             idx_v, idx_all, bufs, tok_buf, gsem, wsem, sem_tok):
    wid = lax.axis_index("s") * 2 + lax.axis_index("c")
    base = wid * BPW

    pltpu.sync_copy(cls1.at[pl.ds(base, BPW)], idx_v)

    tok_g = [pltpu.async_copy(tok.at[idx_v], tok_buf.at[s], sem_tok)
             for s, tok in enumerate((tok_n, tok_p))]

    for h in range(BPW // 16):
        v = idx_v[pl.ds(16 * h, 16)]
        idx_all[0, pl.ds(16 * h, 16)] = v
        for t in range(N_CTX):
            idx_all[1 + t, pl.ds(16 * h, 16)] = v * N_CTX + t
        for j in range(SUF):
            idx_all[1 + N_CTX + j, pl.ds(16 * h, 16)] = v + j * N_CLS

    def table(s, j):
        if j == 0:
            return pre_n if s == 0 else pre_p
        if j <= N_CTX:
            return ctx_n if s == 0 else ctx_p
        return suf_n if s == 0 else suf_p

    tasks = [(s, j) for s in range(2) for j in range(SEQ)]
    nt = len(tasks)
    gh = [None] * nt
    wh = [None] * nt
    for k in range(nt + LOOK):
        if k < nt:
            s, j = tasks[k]
            i = k % NBUF
            if k >= NBUF:
                wh[k - NBUF].wait()
            gh[k] = pltpu.async_copy(table(s, j).at[idx_all.at[j]],
                                     bufs.at[i], gsem.at[i])
        if k >= LOOK:
            m = k - LOOK
            sm, jm = tasks[m]
            gh[m].wait()
            wh[m] = pltpu.async_copy(
                bufs.at[m % NBUF],
                out_t.at[jm, pl.ds(sm * B + base, BPW), :],
                wsem.at[m % NBUF])
    for m in range(nt - NBUF + LOOK, nt):
        wh[m].wait()

    for s in range(2):
        tok_g[s].wait()
        pltpu.async_copy(tok_buf.at[s], out_tok.at[s, pl.ds(base, BPW), :],
                         sem_tok).wait()


_sc_call = functools.partial(
    pl.kernel,
    mesh=plsc.VectorSubcoreMesh(core_axis_name="c", subcore_axis_name="s"),
    out_type=(
        jax.ShapeDtypeStruct((SEQ, 2 * B, CTX_DIM), jnp.float32),
        jax.ShapeDtypeStruct((2, B, TOK_PAD), jnp.int32),
    ),
    scratch_types=[
        pltpu.VMEM((BPW,), jnp.int32),
        pltpu.VMEM((SEQ, BPW), jnp.int32),
        pltpu.VMEM((NBUF, BPW, CTX_DIM), jnp.float32),
        pltpu.VMEM((2, BPW, TOK_PAD), jnp.int32),
        pltpu.SemaphoreType.DMA((NBUF,)),
        pltpu.SemaphoreType.DMA((NBUF,)),
        pltpu.SemaphoreType.DMA,
    ],
)(_sc_body)


def kernel(ctx_pos, ctx_neg, token_prefix_pos, token_suffix_pos,
           token_prefix_neg, token_suffix_neg, tokenized_prompts, cls_id):
    pre_n = token_prefix_neg.reshape(N_CLS, CTX_DIM)
    pre_p = token_prefix_pos.reshape(N_CLS, CTX_DIM)
    ctx_n = ctx_neg.reshape(N_CLS * N_CTX, CTX_DIM)
    ctx_p = ctx_pos.reshape(N_CLS * N_CTX, CTX_DIM)
    suf_n = token_suffix_neg.transpose(1, 0, 2).reshape(SUF * N_CLS, CTX_DIM)
    suf_p = token_suffix_pos.transpose(1, 0, 2).reshape(SUF * N_CLS, CTX_DIM)
    tok_n = jnp.pad(tokenized_prompts[:N_CLS], ((0, 0), (0, TOK_PAD - SEQ)))
    tok_p = jnp.pad(tokenized_prompts[N_CLS:], ((0, 0), (0, TOK_PAD - SEQ)))
    out_t, tok = _sc_call(pre_n, ctx_n, suf_n, pre_p, ctx_p, suf_p,
                          tok_n, tok_p, cls_id)
    return (out_t.transpose(1, 0, 2),
            tok.reshape(2 * B, TOK_PAD)[:, :SEQ])


# speedup vs baseline: 6.3917x; 1.0028x over previous
import functools

import jax
import jax.numpy as jnp
from jax import lax
from jax.experimental import pallas as pl
from jax.experimental.pallas import tpu as pltpu
from jax.experimental.pallas import tpu_sc as plsc

N_CLS = 1000
N_CTX = 16
CTX_DIM = 512
SEQ = 77
SUF = SEQ - 1 - N_CTX
B = 1024
TOK_PAD = 128

NW = 32
BPW = B // NW
NBUF = 6
LOOK = 3


def _sc_body(pre_n, ctx_n, suf_n, pre_p, ctx_p, suf_p, tok_n, tok_p, cls1,
             out_t, out_tok,
             idx_v, idx_all, bufs, tok_buf, gsem, wsem, sem_tok):
    wid = lax.axis_index("s") * 2 + lax.axis_index("c")
    base = wid * BPW

    pltpu.sync_copy(cls1.at[pl.ds(base, BPW)], idx_v)

    tok_g = [pltpu.async_copy(tok.at[idx_v], tok_buf.at[s], sem_tok)
             for s, tok in enumerate((tok_n, tok_p))]

    for h in range(BPW // 16):
        v = idx_v[pl.ds(16 * h, 16)]
        idx_all[0, pl.ds(16 * h, 16)] = v
        for t in range(N_CTX):
            idx_all[1 + t, pl.ds(16 * h, 16)] = v * N_CTX + t
        for j in range(SUF):
            idx_all[1 + N_CTX + j, pl.ds(16 * h, 16)] = v + j * N_CLS

    def table(s, j):
        if j == 0:
            return pre_n if s == 0 else pre_p
        if j <= N_CTX:
            return ctx_n if s == 0 else ctx_p
        return suf_n if s == 0 else suf_p

    tasks = [(s, j) for s in range(2) for j in range(SEQ)]
    nt = len(tasks)
    gh = [None] * nt
    wh = [None] * nt
    for k in range(nt + LOOK):
        if k < nt:
            s, j = tasks[k]
            i = k % NBUF
            if k >= NBUF:
                wh[k - NBUF].wait()
            gh[k] = pltpu.async_copy(table(s, j).at[idx_all.at[j]],
                                     bufs.at[i], gsem.at[i])
        if k >= LOOK:
            m = k - LOOK
            sm, jm = tasks[m]
            gh[m].wait()
            wh[m] = pltpu.async_copy(
                bufs.at[m % NBUF],
                out_t.at[jm, pl.ds(sm * B + base, BPW), :],
                wsem.at[m % NBUF])
    for m in range(nt - NBUF + LOOK, nt):
        wh[m].wait()

    for s in range(2):
        tok_g[s].wait()
        pltpu.async_copy(tok_buf.at[s], out_tok.at[s, pl.ds(base, BPW), :],
                         sem_tok).wait()


_sc_call = functools.partial(
    pl.kernel,
    mesh=plsc.VectorSubcoreMesh(core_axis_name="c", subcore_axis_name="s"),
    out_type=(
        jax.ShapeDtypeStruct((SEQ, 2 * B, CTX_DIM), jnp.float32),
        jax.ShapeDtypeStruct((2, B, TOK_PAD), jnp.int32),
    ),
    scratch_types=[
        pltpu.VMEM((BPW,), jnp.int32),
        pltpu.VMEM((SEQ, BPW), jnp.int32),
        pltpu.VMEM((NBUF, BPW, CTX_DIM), jnp.float32),
        pltpu.VMEM((2, BPW, TOK_PAD), jnp.int32),
        pltpu.SemaphoreType.DMA((NBUF,)),
        pltpu.SemaphoreType.DMA((NBUF,)),
        pltpu.SemaphoreType.DMA,
    ],
)(_sc_body)


def kernel(ctx_pos, ctx_neg, token_prefix_pos, token_suffix_pos,
           token_prefix_neg, token_suffix_neg, tokenized_prompts, cls_id):
    pre_n = token_prefix_neg.reshape(N_CLS, CTX_DIM)
    pre_p = token_prefix_pos.reshape(N_CLS, CTX_DIM)
    ctx_n = ctx_neg.reshape(N_CLS * N_CTX, CTX_DIM)
    ctx_p = ctx_pos.reshape(N_CLS * N_CTX, CTX_DIM)
    suf_n = token_suffix_neg.transpose(1, 0, 2).reshape(SUF * N_CLS, CTX_DIM)
    suf_p = token_suffix_pos.transpose(1, 0, 2).reshape(SUF * N_CLS, CTX_DIM)
    tok_n = jnp.pad(tokenized_prompts[:N_CLS], ((0, 0), (0, TOK_PAD - SEQ)))
    tok_p = jnp.pad(tokenized_prompts[N_CLS:], ((0, 0), (0, TOK_PAD - SEQ)))
    out_t, tok = _sc_call(pre_n, ctx_n, suf_n, pre_p, ctx_p, suf_p,
                          tok_n, tok_p, cls_id)
    return (out_t.transpose(1, 0, 2),
            tok.reshape(2 * B, TOK_PAD)[:, :SEQ])


# speedup vs baseline: 6.4393x; 1.0074x over previous
import functools

import jax
import jax.numpy as jnp
from jax import lax
from jax.experimental import pallas as pl
from jax.experimental.pallas import tpu as pltpu
from jax.experimental.pallas import tpu_sc as plsc

N_CLS = 1000
N_CTX = 16
CTX_DIM = 512
SEQ = 77
SUF = SEQ - 1 - N_CTX
B = 1024
TOK_PAD = 128

NW = 32
BPW = B // NW
NPAIR = (SEQ - 1) // 2
NBUF = 3
LOOK = 2


def _sc_body(pre_n, ctx_n, suf_n, pre_p, ctx_p, suf_p, tok_n, tok_p, cls1,
             out_t, out_tok,
             idx_v, idx_all, bufs, tok_buf, gsem, wsem, sem_tok):
    wid = lax.axis_index("s") * 2 + lax.axis_index("c")
    base = wid * BPW

    pltpu.sync_copy(cls1.at[pl.ds(base, BPW)], idx_v)

    tok_g = [pltpu.async_copy(tok.at[idx_v], tok_buf.at[s], sem_tok)
             for s, tok in enumerate((tok_n, tok_p))]

    def slab_pair(p):
        if p < N_CTX // 2:
            return 1 + 2 * p
        return 1 + N_CTX + 2 * (p - N_CTX // 2)

    def flat_index(v, j):
        if j == 0:
            return v
        if j <= N_CTX:
            return v * N_CTX + (j - 1)
        return (j - 1 - N_CTX) * N_CLS + v

    for p in range(NPAIR):
        j0 = slab_pair(p)
        for d in range(2):
            for h in range(BPW // 16):
                v = idx_v[pl.ds(16 * h, 16)]
                idx_all[p, pl.ds(32 * d + 16 * h, 16)] = flat_index(v, j0 + d)

    tasks = [(s, None) for s in range(2)] + \
            [(s, p) for s in range(2) for p in range(NPAIR)]
    nt = len(tasks)

    def tables(s):
        return (pre_n, ctx_n, suf_n) if s == 0 else (pre_p, ctx_p, suf_p)

    def start_gather(k, i):
        s, p = tasks[k]
        if p is None:
            return pltpu.async_copy(tables(s)[0].at[idx_v],
                                    bufs.at[i, pl.ds(0, BPW)], gsem.at[i])
        tab = tables(s)[1] if slab_pair(p) <= N_CTX else tables(s)[2]
        return pltpu.async_copy(tab.at[idx_all.at[p]], bufs.at[i], gsem.at[i])

    def start_write(k, i):
        s, p = tasks[k]
        if p is None:
            return [pltpu.async_copy(
                bufs.at[i, pl.ds(0, BPW)],
                out_t.at[0, pl.ds(s * B + base, BPW), :], wsem.at[i])]
        j0 = slab_pair(p)
        return [pltpu.async_copy(
            bufs.at[i, pl.ds(d * BPW, BPW)],
            out_t.at[j0 + d, pl.ds(s * B + base, BPW), :], wsem.at[i])
            for d in range(2)]

    gh = [None] * nt
    wh = [None] * nt
    for k in range(nt + LOOK):
        if k < nt:
            i = k % NBUF
            if k >= NBUF:
                for w in wh[k - NBUF]:
                    w.wait()
            gh[k] = start_gather(k, i)
        if k >= LOOK:
            m = k - LOOK
            gh[m].wait()
            wh[m] = start_write(m, m % NBUF)
    for m in range(nt - NBUF + LOOK, nt):
        for w in wh[m]:
            w.wait()

    for s in range(2):
        tok_g[s].wait()
        pltpu.async_copy(tok_buf.at[s], out_tok.at[s, pl.ds(base, BPW), :],
                         sem_tok).wait()


_sc_call = functools.partial(
    pl.kernel,
    mesh=plsc.VectorSubcoreMesh(core_axis_name="c", subcore_axis_name="s"),
    out_type=(
        jax.ShapeDtypeStruct((SEQ, 2 * B, CTX_DIM), jnp.float32),
        jax.ShapeDtypeStruct((2, B, TOK_PAD), jnp.int32),
    ),
    scratch_types=[
        pltpu.VMEM((BPW,), jnp.int32),
        pltpu.VMEM((NPAIR, 2 * BPW), jnp.int32),
        pltpu.VMEM((NBUF, 2 * BPW, CTX_DIM), jnp.float32),
        pltpu.VMEM((2, BPW, TOK_PAD), jnp.int32),
        pltpu.SemaphoreType.DMA((NBUF,)),
        pltpu.SemaphoreType.DMA((NBUF,)),
        pltpu.SemaphoreType.DMA,
    ],
)(_sc_body)


def kernel(ctx_pos, ctx_neg, token_prefix_pos, token_suffix_pos,
           token_prefix_neg, token_suffix_neg, tokenized_prompts, cls_id):
    pre_n = token_prefix_neg.reshape(N_CLS, CTX_DIM)
    pre_p = token_prefix_pos.reshape(N_CLS, CTX_DIM)
    ctx_n = ctx_neg.reshape(N_CLS * N_CTX, CTX_DIM)
    ctx_p = ctx_pos.reshape(N_CLS * N_CTX, CTX_DIM)
    suf_n = token_suffix_neg.transpose(1, 0, 2).reshape(SUF * N_CLS, CTX_DIM)
    suf_p = token_suffix_pos.transpose(1, 0, 2).reshape(SUF * N_CLS, CTX_DIM)
    tok_n = jnp.pad(tokenized_prompts[:N_CLS], ((0, 0), (0, TOK_PAD - SEQ)))
    tok_p = jnp.pad(tokenized_prompts[N_CLS:], ((0, 0), (0, TOK_PAD - SEQ)))
    out_t, tok = _sc_call(pre_n, ctx_n, suf_n, pre_p, ctx_p, suf_p,
                          tok_n, tok_p, cls_id)
    return (out_t.transpose(1, 0, 2),
            tok.reshape(2 * B, TOK_PAD)[:, :SEQ])


# speedup vs baseline: 6.5443x; 1.0163x over previous
import functools

import jax
import jax.numpy as jnp
from jax import lax
from jax.experimental import pallas as pl
from jax.experimental.pallas import tpu as pltpu
from jax.experimental.pallas import tpu_sc as plsc

N_CLS = 1000
N_CTX = 16
CTX_DIM = 512
SEQ = 77
SUF = SEQ - 1 - N_CTX
B = 1024
TOK_PAD = 128

NW = 32
BPW = B // NW
NPAIR = (SEQ - 1) // 2
NBUF = 3
LOOK = 2


def _sc_body(pre_n, ctx_n, suf_n, pre_p, ctx_p, suf_p, tok_n, tok_p, cls1,
             out_t, out_tok,
             idx_v, idx_all, bufs, tok_buf, gsem, wsem, sem_tok):
    wid = lax.axis_index("s") * 2 + lax.axis_index("c")
    base = wid * BPW

    pltpu.sync_copy(cls1.at[pl.ds(base, BPW)], idx_v)

    tok_g = [pltpu.async_copy(tok.at[idx_v], tok_buf.at[s], sem_tok)
             for s, tok in enumerate((tok_n, tok_p))]

    def slab_pair(p):
        if p < N_CTX // 2:
            return 1 + 2 * p
        return 1 + N_CTX + 2 * (p - N_CTX // 2)

    def flat_index(v, j):
        if j == 0:
            return v
        if j <= N_CTX:
            return v * N_CTX + (j - 1)
        return (j - 1 - N_CTX) * N_CLS + v

    tasks = [(s, None) for s in range(2)] + \
            [(s, p) for s in range(2) for p in range(NPAIR)]
    nt = len(tasks)

    def tables(s):
        return (pre_n, ctx_n, suf_n) if s == 0 else (pre_p, ctx_p, suf_p)

    def start_gather(k, i):
        s, p = tasks[k]
        if p is None:
            return pltpu.async_copy(tables(s)[0].at[idx_v],
                                    bufs.at[i, pl.ds(0, BPW)], gsem.at[i])
        tab = tables(s)[1] if slab_pair(p) <= N_CTX else tables(s)[2]
        return pltpu.async_copy(tab.at[idx_all.at[p]], bufs.at[i], gsem.at[i])

    def start_write(k, i):
        s, p = tasks[k]
        if p is None:
            return [pltpu.async_copy(
                bufs.at[i, pl.ds(0, BPW)],
                out_t.at[0, pl.ds(s * B + base, BPW), :], wsem.at[i])]
        j0 = slab_pair(p)
        return [pltpu.async_copy(
            bufs.at[i, pl.ds(d * BPW, BPW)],
            out_t.at[j0 + d, pl.ds(s * B + base, BPW), :], wsem.at[i])
            for d in range(2)]

    gh = [None] * nt
    wh = [None] * nt

    for k in range(2):
        gh[k] = start_gather(k, k % NBUF)
    for p in range(NPAIR):
        j0 = slab_pair(p)
        for d in range(2):
            for h in range(BPW // 16):
                v = idx_v[pl.ds(16 * h, 16)]
                idx_all[p, pl.ds(32 * d + 16 * h, 16)] = flat_index(v, j0 + d)

    for k in range(2, nt + LOOK):
        if k < nt:
            i = k % NBUF
            if k >= NBUF:
                for w in wh[k - NBUF]:
                    w.wait()
            gh[k] = start_gather(k, i)
        if k >= LOOK:
            m = k - LOOK
            gh[m].wait()
            wh[m] = start_write(m, m % NBUF)
    for m in range(nt - NBUF + LOOK, nt):
        for w in wh[m]:
            w.wait()

    for s in range(2):
        tok_g[s].wait()
        pltpu.async_copy(tok_buf.at[s], out_tok.at[s, pl.ds(base, BPW), :],
                         sem_tok).wait()


_sc_call = functools.partial(
    pl.kernel,
    mesh=plsc.VectorSubcoreMesh(core_axis_name="c", subcore_axis_name="s"),
    out_type=(
        jax.ShapeDtypeStruct((SEQ, 2 * B, CTX_DIM), jnp.float32),
        jax.ShapeDtypeStruct((2, B, TOK_PAD), jnp.int32),
    ),
    scratch_types=[
        pltpu.VMEM((BPW,), jnp.int32),
        pltpu.VMEM((NPAIR, 2 * BPW), jnp.int32),
        pltpu.VMEM((NBUF, 2 * BPW, CTX_DIM), jnp.float32),
        pltpu.VMEM((2, BPW, TOK_PAD), jnp.int32),
        pltpu.SemaphoreType.DMA((NBUF,)),
        pltpu.SemaphoreType.DMA((NBUF,)),
        pltpu.SemaphoreType.DMA,
    ],
)(_sc_body)


def kernel(ctx_pos, ctx_neg, token_prefix_pos, token_suffix_pos,
           token_prefix_neg, token_suffix_neg, tokenized_prompts, cls_id):
    pre_n = token_prefix_neg.reshape(N_CLS, CTX_DIM)
    pre_p = token_prefix_pos.reshape(N_CLS, CTX_DIM)
    ctx_n = ctx_neg.reshape(N_CLS * N_CTX, CTX_DIM)
    ctx_p = ctx_pos.reshape(N_CLS * N_CTX, CTX_DIM)
    suf_n = token_suffix_neg.transpose(1, 0, 2).reshape(SUF * N_CLS, CTX_DIM)
    suf_p = token_suffix_pos.transpose(1, 0, 2).reshape(SUF * N_CLS, CTX_DIM)
    tok_n = jnp.pad(tokenized_prompts[:N_CLS], ((0, 0), (0, TOK_PAD - SEQ)))
    tok_p = jnp.pad(tokenized_prompts[N_CLS:], ((0, 0), (0, TOK_PAD - SEQ)))
    out_t, tok = _sc_call(pre_n, ctx_n, suf_n, pre_p, ctx_p, suf_p,
                          tok_n, tok_p, cls_id)
    return (out_t.transpose(1, 0, 2),
            tok.reshape(2 * B, TOK_PAD)[:, :SEQ])
